# baseline (speedup 1.0000x reference)
import functools

import jax
import jax.numpy as jnp
from jax import lax
from jax.experimental import pallas as pl
from jax.experimental.pallas import tpu as pltpu
from jax.experimental.pallas import tpu_sc as plsc

D = 128
L = 16
NVR = D // L
NC, NS = 2, 16
NW = NC * NS

N = 10000
E = 160000
MN = 100
ME = 400
AW = D + L

INV_SQRT_D = 1.0 / (D ** 0.5)

_SC_MESH = dict(core_axis_name="c", subcore_axis_name="s",
                num_cores=NC, num_subcores=NS)


def _emb_gather(table, idx):
  B = idx.shape[0]
  b_per_w = B // NW
  CH = 80
  n_ch = b_per_w // CH
  mesh = plsc.VectorSubcoreMesh(**_SC_MESH)

  @functools.partial(
      pl.kernel, mesh=mesh,
      out_type=jax.ShapeDtypeStruct((B, D), jnp.float32),
      compiler_params=pltpu.CompilerParams(needs_layout_passes=False),
      scratch_types=[
          pltpu.VMEM((CH,), jnp.int32),
          pltpu.VMEM((CH, D), jnp.float32),
          pltpu.SemaphoreType.DMA,
      ])
  def gk(table_hbm, idx_hbm, out_hbm, idx_v, rows_v, sem):
    wid = lax.axis_index("s") * NC + lax.axis_index("c")

    def body(c, carry):
      base = wid * b_per_w + c * CH
      pltpu.sync_copy(idx_hbm.at[pl.ds(base, CH)], idx_v)
      pltpu.async_copy(table_hbm.at[idx_v], rows_v, sem).wait()
      pltpu.sync_copy(rows_v, out_hbm.at[pl.ds(base, CH)])
      return carry

    lax.fori_loop(0, n_ch, body, 0)

  return gk(table, idx)


DROWS = 80


def _edge_pass(q, kv, w21, srcme, dst):
  C = 40
  NCH_TOT = E // C
  ZR = 40
  NZC = N // ZR
  mesh = plsc.VectorSubcoreMesh(**_SC_MESH)

  @functools.partial(
      pl.kernel, mesh=mesh,
      out_type=[jax.ShapeDtypeStruct((NC * N, D), jnp.float32),
                jax.ShapeDtypeStruct((NC * DROWS, D), jnp.float32)],
      compiler_params=pltpu.CompilerParams(needs_layout_passes=False),
      scratch_types=[
          pltpu.VMEM_SHARED((N, D), jnp.float32),
          pltpu.VMEM_SHARED((DROWS, D), jnp.float32),
          pltpu.VMEM((C,), jnp.int32),
          pltpu.VMEM((C,), jnp.int32),
          pltpu.VMEM((C,), jnp.int32),
          pltpu.VMEM((C, D), jnp.float32),
          pltpu.VMEM((C, 2 * D), jnp.float32),
          pltpu.VMEM((C, 2 * D), jnp.float32),
          pltpu.VMEM((C, D), jnp.float32),
          pltpu.VMEM((DROWS, D), jnp.float32),
          pltpu.VMEM((DROWS,), jnp.int32),
          pltpu.VMEM((L * L,), jnp.float32),
          pltpu.VMEM((L,), jnp.float32),
          pltpu.SemaphoreType.DMA,
      ])
  def ek(q_h, kv_h, w_h, srcme_h, dst_h, out_h, outd_h,
         acc_sh, accd_sh, dst_v, src_v, me_v, q_r, kv_r, w_r,
         msg_r, den_r, idn_r, red_r, res_r, sem):
    cid = lax.axis_index("c")
    sid = lax.axis_index("s")
    wid = sid * NC + cid
    iota16 = lax.iota(jnp.int32, L)
    zv = jnp.zeros((L,), jnp.float32)

    for r in range(ZR):
      for d in range(D // L):
        msg_r[r, pl.ds(d * L, L)] = zv
    for g in range(DROWS // L):
      idn_r[pl.ds(g * L, L)] = iota16 + g * L
    for r in range(DROWS):
      for d in range(D // L):
        den_r[r, pl.ds(d * L, L)] = zv

    n_my_zero = (NZC - 1 - sid) // NS + 1

    def zb(i, carry):
      r = (sid + i * NS) * ZR
      pltpu.sync_copy(msg_r, acc_sh.at[pl.ds(r, ZR)])
      return carry

    lax.fori_loop(0, n_my_zero, zb, 0)

    @pl.when(sid == 0)
    def _():
      pltpu.sync_copy(msg_r, accd_sh.at[pl.ds(0, ZR)])
      pltpu.sync_copy(msg_r, accd_sh.at[pl.ds(ZR, ZR)])

    plsc.subcore_barrier()

    n_my_chunks = (NCH_TOT - 1 - wid) // NW + 1

    def chunk(c, carry):
      base = (wid + c * NW) * C
      cpi = [
          pltpu.async_copy(dst_h.at[pl.ds(base, C)], dst_v, sem),
          pltpu.async_copy(srcme_h.at[pl.ds(2 * base, C)], src_v, sem),
          pltpu.async_copy(srcme_h.at[pl.ds(2 * base + C, C)], me_v, sem),
      ]
      for cp in cpi:
        cp.wait()
      cps = [
          pltpu.async_copy(q_h.at[dst_v], q_r, sem),
          pltpu.async_copy(kv_h.at[src_v], kv_r, sem),
          pltpu.async_copy(w_h.at[me_v], w_r, sem),
      ]
      for cp in cps:
        cp.wait()

      for goff, ioff, lb, glen in ((0, 0, 0, L), (16, 16, 0, L),
                                   (32, 24, 8, 8)):
        def edge_dot(j, carry2):
          i = goff + j
          acc = jnp.zeros((L,), jnp.float32)
          for d in range(NVR):
            sl = pl.ds(d * L, L)
            sl2 = pl.ds(D + d * L, L)
            acc = acc + q_r[i, sl] * (
                kv_r[i, sl] * (1.0 + w_r[i, sl]) + w_r[i, sl2])
          red_r[pl.ds((lb + j) * L, L)] = acc
          return carry2

        lax.fori_loop(0, glen, edge_dot, 0)
        res = jnp.zeros((L,), jnp.float32)
        for jj in range(L):
          res = res + plsc.load_gather(red_r, [iota16 * L + jj])
        ev16 = jnp.exp(res * INV_SQRT_D)
        res_r[pl.ds(0, L)] = ev16

        def edge_msg(j, carry2):
          i = goff + j
          ev = plsc.load_gather(
              res_r, [jnp.broadcast_to(lb + j, (L,)).astype(jnp.int32)])
          for d in range(NVR):
            sl = pl.ds(d * L, L)
            sl2 = pl.ds(D + d * L, L)
            msg_r[i, sl] = ev * (kv_r[i, sl2] * (1.0 + w_r[i, sl]))
          return carry2

        lax.fori_loop(0, glen, edge_msg, 0)
        dstg = dst_v[pl.ds(ioff, L)]
        rowv = lax.shift_right_logical(dstg, 7)
        colv = jnp.bitwise_and(dstg, 127)
        for j in range(lb, L):
          plsc.addupdate_scatter(den_r, [rowv, colv], ev16,
                                 mask=iota16 == j)

      pltpu.sync_copy(msg_r, acc_sh.at[dst_v], add=True)
      return carry

    lax.fori_loop(0, n_my_chunks, chunk, 0)

    pltpu.sync_copy(den_r, accd_sh.at[idn_r], add=True)
    plsc.subcore_barrier()

    def ob(i, carry):
      r = (sid + i * NS) * ZR
      pltpu.sync_copy(acc_sh.at[pl.ds(r, ZR)],
                      out_h.at[pl.ds(cid * N + r, ZR)])
      return carry

    lax.fori_loop(0, n_my_zero, ob, 0)

    @pl.when(sid == 0)
    def _():
      pltpu.sync_copy(accd_sh, outd_h.at[pl.ds(cid * DROWS, DROWS)])

  return ek(q, kv, w21, srcme, dst)


def _meta_block(mnf, mei, wq, wk, wv, wo, wm1, wm2):
  def body(mei_ref, mnf_ref, wq_ref, wk_ref, wv_ref, wo_ref, wm1_ref,
           wm2_ref, np2_ref, np1_ref, ep21_ref):
    f = mnf_ref[...]
    iota_n = lax.broadcasted_iota(jnp.int32, (ME, MN), 1)
    msrc = mei_ref[0, :].reshape(ME, 1)
    mdst = mei_ref[1, :].reshape(ME, 1)
    oh_s = (msrc == iota_n).astype(jnp.float32)
    oh_d = (mdst == iota_n).astype(jnp.float32)
    mm = functools.partial(jnp.dot, preferred_element_type=jnp.float32)
    q = mm(f, wq_ref[...])
    kk = mm(f, wk_ref[...])
    vv = mm(f, wv_ref[...])
    q_d = mm(oh_d, q)
    k_e = mm(oh_s, kk)
    v_e = mm(oh_s, vv)
    logits = jnp.sum(q_d * k_e, axis=1, keepdims=True) * INV_SQRT_D
    masked = jnp.where(oh_d > 0.0, logits, -1e30)
    m = jnp.max(masked, axis=0, keepdims=True)
    m = jnp.where(m < -1e29, 0.0, m)
    ex = jnp.exp(logits - mm(oh_d, m.reshape(MN, 1)))
    ssum = lax.dot_general(oh_d, ex, (((0,), (0,)), ((), ())),
                           preferred_element_type=jnp.float32)
    alpha = ex / (mm(oh_d, ssum) + 1e-9)
    msg = alpha * v_e
    segsum = lax.dot_general(oh_d, msg, (((0,), (0,)), ((), ())),
                             preferred_element_type=jnp.float32)
    mn_out = jnp.maximum(mm(segsum, wo_ref[...]), 0.0)
    np2_ref[...] = jnp.tanh(mm(mn_out, wm2_ref[...]))
    np1_ref[...] = jnp.tanh(mm(mn_out, wm1_ref[...]))
    ep21_ref[:, :D] = jnp.tanh(mm(msg, wm2_ref[...]))
    ep21_ref[:, D:] = jnp.tanh(mm(msg, wm1_ref[...]))

  out = [jax.ShapeDtypeStruct((MN, D), jnp.float32),
         jax.ShapeDtypeStruct((MN, D), jnp.float32),
         jax.ShapeDtypeStruct((ME, 2 * D), jnp.float32)]
  return pl.pallas_call(body, out_shape=out)(
      mei, mnf, wq, wk, wv, wo, wm1, wm2)


_TILE = 400
_NT = N // _TILE


def _node_params(mid3, np2, np1):
  def body(mid_ref, np2_ref, np1_ref, w2_ref, w1_ref):
    mid = mid_ref[0, 0, :].reshape(_TILE, 1)
    iota_n = lax.broadcasted_iota(jnp.int32, (_TILE, MN), 1)
    oh = (mid == iota_n).astype(jnp.float32)
    mm = functools.partial(jnp.dot, preferred_element_type=jnp.float32)
    w2_ref[...] = mm(oh, np2_ref[...])
    w1_ref[...] = mm(oh, np1_ref[...])

  out = [jax.ShapeDtypeStruct((N, D), jnp.float32),
         jax.ShapeDtypeStruct((N, D), jnp.float32)]
  return pl.pallas_call(
      body,
      grid=(_NT,),
      in_specs=[pl.BlockSpec((1, 1, _TILE), lambda i: (i, 0, 0)),
                pl.BlockSpec((MN, D), lambda i: (0, 0)),
                pl.BlockSpec((MN, D), lambda i: (0, 0))],
      out_specs=[pl.BlockSpec((_TILE, D), lambda i: (i, 0)),
                 pl.BlockSpec((_TILE, D), lambda i: (i, 0))],
      out_shape=out)(mid3, np2, np1)


def _qkv(feat, wq, wk, wv, w2n):
  def body(f_ref, wq_ref, wk_ref, wv_ref, w2_ref, q_ref, kv_ref):
    f = f_ref[...]
    mm = functools.partial(jnp.dot, preferred_element_type=jnp.float32)
    q_ref[...] = mm(f, wq_ref[...]) * (1.0 + w2_ref[...])
    kv_ref[:, :D] = mm(f, wk_ref[...])
    kv_ref[:, D:] = mm(f, wv_ref[...])

  out = [jax.ShapeDtypeStruct((N, D), jnp.float32),
         jax.ShapeDtypeStruct((N, 2 * D), jnp.float32)]
  tile = pl.BlockSpec((_TILE, D), lambda i: (i, 0))
  tile2 = pl.BlockSpec((_TILE, 2 * D), lambda i: (i, 0))
  full = pl.BlockSpec((D, D), lambda i: (0, 0))
  return pl.pallas_call(
      body, grid=(_NT,),
      in_specs=[tile, full, full, full, tile],
      out_specs=[tile, tile2],
      out_shape=out)(feat, wq, wk, wv, w2n)


def _combine(acc0, acc1, den0_3, den1_3, wo, w1n):
  def body(a0_ref, a1_ref, d0_ref, d1_ref, wo_ref, w1_ref, o_ref):
    num = a0_ref[...] + a1_ref[...]
    den = (d0_ref[0, 0, :] + d1_ref[0, 0, :]).reshape(_TILE, 1)
    pre = num / (den + 1e-9)
    mm = functools.partial(jnp.dot, preferred_element_type=jnp.float32)
    o_ref[...] = jnp.maximum(mm(pre, wo_ref[...]) + w1_ref[...], 0.0)

  tile = pl.BlockSpec((_TILE, D), lambda i: (i, 0))
  dtile = pl.BlockSpec((1, 1, _TILE), lambda i: (i, 0, 0))
  full = pl.BlockSpec((D, D), lambda i: (0, 0))
  return pl.pallas_call(
      body, grid=(_NT,),
      in_specs=[tile, tile, dtile, dtile, full, tile],
      out_specs=tile,
      out_shape=jax.ShapeDtypeStruct((N, D), jnp.float32))(
          acc0, acc1, den0_3, den1_3, wo, w1n)


def kernel(node_values, edge_index, meta_node_values, meta_edge_index,
           meta_node_id, meta_edge_id, emb_table, Wm1, Wm2, Wq_meta,
           Wk_meta, Wv_meta, Wo_meta, Wq, Wk, Wv, Wo):
  nv = node_values.astype(jnp.int32)
  mnv = meta_node_values.astype(jnp.int32)
  total = N + MN
  padded = ((total + 8 * NW * 10 - 1) // (8 * NW * 10)) * (8 * NW * 10)
  idx_all = jnp.concatenate(
      [nv, mnv, jnp.zeros((padded - total,), jnp.int32)])
  rows = _emb_gather(emb_table.astype(jnp.float32), idx_all)
  node_feat = rows[:N]
  mn_feat = rows[N:N + MN]

  np2, np1, ep21 = _meta_block(
      mn_feat, meta_edge_index.astype(jnp.int32),
      Wq_meta[0], Wk_meta[0], Wv_meta[0], Wo_meta[0], Wm1, Wm2)

  mid3 = meta_node_id.astype(jnp.int32).reshape(_NT, 1, _TILE)
  w2n, w1n = _node_params(mid3, np2, np1)

  src = edge_index[0].astype(jnp.int32)
  dst = edge_index[1].astype(jnp.int32)
  me = meta_edge_id.astype(jnp.int32)
  srcme = jnp.concatenate(
      [src.reshape(-1, 40), me.reshape(-1, 40)], axis=1).reshape(-1)

  feats = []
  feat = node_feat
  for i in range(2):
    q, kv = _qkv(feat, Wq[i], Wk[i], Wv[i], w2n)
    acc, accd = _edge_pass(q, kv, ep21, srcme, dst)
    den0_3 = accd[:DROWS].reshape(DROWS * D)[:N].reshape(_NT, 1, _TILE)
    den1_3 = accd[DROWS:].reshape(DROWS * D)[:N].reshape(_NT, 1, _TILE)
    feat = _combine(acc[:N], acc[N:], den0_3, den1_3, Wo[i], w1n)
    feats.append(feat)
  return jnp.stack(feats)


# speedup vs baseline: 1.0158x; 1.0158x over previous
import functools

import jax
import jax.numpy as jnp
from jax import lax
from jax.experimental import pallas as pl
from jax.experimental.pallas import tpu as pltpu
from jax.experimental.pallas import tpu_sc as plsc

D = 128
L = 16
NVR = D // L
NC, NS = 2, 16
NW = NC * NS

N = 10000
E = 160000
MN = 100
ME = 400
AW = D + L

INV_SQRT_D = 1.0 / (D ** 0.5)

_SC_MESH = dict(core_axis_name="c", subcore_axis_name="s",
                num_cores=NC, num_subcores=NS)


def _emb_gather(table, idx):
  B = idx.shape[0]
  b_per_w = B // NW
  CH = 80
  n_ch = b_per_w // CH
  mesh = plsc.VectorSubcoreMesh(**_SC_MESH)

  @functools.partial(
      pl.kernel, mesh=mesh,
      out_type=jax.ShapeDtypeStruct((B, D), jnp.float32),
      compiler_params=pltpu.CompilerParams(needs_layout_passes=False),
      scratch_types=[
          pltpu.VMEM((CH,), jnp.int32),
          pltpu.VMEM((CH, D), jnp.float32),
          pltpu.SemaphoreType.DMA,
      ])
  def gk(table_hbm, idx_hbm, out_hbm, idx_v, rows_v, sem):
    wid = lax.axis_index("s") * NC + lax.axis_index("c")

    def body(c, carry):
      base = wid * b_per_w + c * CH
      pltpu.sync_copy(idx_hbm.at[pl.ds(base, CH)], idx_v)
      pltpu.async_copy(table_hbm.at[idx_v], rows_v, sem).wait()
      pltpu.sync_copy(rows_v, out_hbm.at[pl.ds(base, CH)])
      return carry

    lax.fori_loop(0, n_ch, body, 0)

  return gk(table, idx)


DROWS = 80


def _edge_pass(q, kv, w21, srcme, dst):
  C = 40
  NCH_TOT = E // C
  ZR = 40
  NZC = N // ZR
  mesh = plsc.VectorSubcoreMesh(**_SC_MESH)

  @functools.partial(
      pl.kernel, mesh=mesh,
      out_type=[jax.ShapeDtypeStruct((NC * N, D), jnp.float32),
                jax.ShapeDtypeStruct((NC * DROWS, D), jnp.float32)],
      compiler_params=pltpu.CompilerParams(needs_layout_passes=False),
      scratch_types=[
          pltpu.VMEM_SHARED((N, D), jnp.float32),
          pltpu.VMEM_SHARED((DROWS, D), jnp.float32),
          pltpu.VMEM((C,), jnp.int32),
          pltpu.VMEM((C,), jnp.int32),
          pltpu.VMEM((C,), jnp.int32),
          pltpu.VMEM((C,), jnp.int32),
          pltpu.VMEM((C, D), jnp.float32),
          pltpu.VMEM((C, 2 * D), jnp.float32),
          pltpu.VMEM((C, 2 * D), jnp.float32),
          pltpu.VMEM((C, D), jnp.float32),
          pltpu.VMEM((DROWS, D), jnp.float32),
          pltpu.VMEM((DROWS,), jnp.int32),
          pltpu.VMEM((L * L,), jnp.float32),
          pltpu.VMEM((L,), jnp.float32),
          pltpu.SemaphoreType.DMA,
          pltpu.SemaphoreType.DMA,
      ])
  def ek(q_h, kv_h, w_h, srcme_h, dst_h, out_h, outd_h,
         acc_sh, accd_sh, dst_a, dst_b, src_v, me_v, q_r, kv_r, w_r,
         msg_r, den_r, idn_r, red_r, res_r, sem, sem2):
    cid = lax.axis_index("c")
    sid = lax.axis_index("s")
    wid = sid * NC + cid
    iota16 = lax.iota(jnp.int32, L)
    zv = jnp.zeros((L,), jnp.float32)

    for r in range(ZR):
      for d in range(D // L):
        msg_r[r, pl.ds(d * L, L)] = zv
    for g in range(DROWS // L):
      idn_r[pl.ds(g * L, L)] = iota16 + g * L
    for r in range(DROWS):
      for d in range(D // L):
        den_r[r, pl.ds(d * L, L)] = zv

    n_my_zero = (NZC - 1 - sid) // NS + 1

    def zb(i, carry):
      r = (sid + i * NS) * ZR
      pltpu.sync_copy(msg_r, acc_sh.at[pl.ds(r, ZR)])
      return carry

    lax.fori_loop(0, n_my_zero, zb, 0)

    @pl.when(sid == 0)
    def _():
      pltpu.sync_copy(msg_r, accd_sh.at[pl.ds(0, ZR)])
      pltpu.sync_copy(msg_r, accd_sh.at[pl.ds(ZR, ZR)])

    plsc.subcore_barrier()

    n_my_chunks = (NCH_TOT - 1 - wid) // NW + 1

    def chunk_body(c, dst_v):
      base = (wid + c * NW) * C
      cpi = [
          pltpu.async_copy(dst_h.at[pl.ds(base, C)], dst_v, sem),
          pltpu.async_copy(srcme_h.at[pl.ds(2 * base, C)], src_v, sem),
          pltpu.async_copy(srcme_h.at[pl.ds(2 * base + C, C)], me_v, sem),
      ]
      for cp in cpi:
        cp.wait()
      cps = [
          pltpu.async_copy(q_h.at[dst_v], q_r, sem),
          pltpu.async_copy(kv_h.at[src_v], kv_r, sem),
          pltpu.async_copy(w_h.at[me_v], w_r, sem),
      ]
      for cp in cps:
        cp.wait()

      @pl.when(c > 0)
      def _():
        pltpu.make_async_copy(out_h.at[pl.ds(0, C)], msg_r, sem2).wait()

      for goff, ioff, lb, glen in ((0, 0, 0, L), (16, 16, 0, L),
                                   (32, 24, 8, 8)):
        def edge_dot(j, carry2):
          i = goff + j
          acc = jnp.zeros((L,), jnp.float32)
          for d in range(NVR):
            sl = pl.ds(d * L, L)
            sl2 = pl.ds(D + d * L, L)
            acc = acc + q_r[i, sl] * (
                kv_r[i, sl] * (1.0 + w_r[i, sl]) + w_r[i, sl2])
          red_r[pl.ds((lb + j) * L, L)] = acc
          return carry2

        lax.fori_loop(0, glen, edge_dot, 0)
        res = jnp.zeros((L,), jnp.float32)
        for jj in range(L):
          res = res + plsc.load_gather(red_r, [iota16 * L + jj])
        ev16 = jnp.exp(res * INV_SQRT_D)
        res_r[pl.ds(0, L)] = ev16

        def edge_msg(j, carry2):
          i = goff + j
          ev = plsc.load_gather(
              res_r, [jnp.broadcast_to(lb + j, (L,)).astype(jnp.int32)])
          for d in range(NVR):
            sl = pl.ds(d * L, L)
            sl2 = pl.ds(D + d * L, L)
            msg_r[i, sl] = ev * (kv_r[i, sl2] * (1.0 + w_r[i, sl]))
          return carry2

        lax.fori_loop(0, glen, edge_msg, 0)
        dstg = dst_v[pl.ds(ioff, L)]
        rowv = lax.shift_right_logical(dstg, 7)
        colv = jnp.bitwise_and(dstg, 127)
        for j in range(lb, L):
          plsc.addupdate_scatter(den_r, [rowv, colv], ev16,
                                 mask=iota16 == j)

      pltpu.async_copy(msg_r, acc_sh.at[dst_v], sem2, add=True)

    def pair(i, carry):
      chunk_body(2 * i, dst_a)
      chunk_body(2 * i + 1, dst_b)
      return carry

    lax.fori_loop(0, n_my_chunks // 2, pair, 0)
    chunk_body(n_my_chunks - 1, dst_a)
    pltpu.make_async_copy(out_h.at[pl.ds(0, C)], msg_r, sem2).wait()

    pltpu.sync_copy(den_r, accd_sh.at[idn_r], add=True)
    plsc.subcore_barrier()

    def ob(i, carry):
      r = (sid + i * NS) * ZR
      pltpu.sync_copy(acc_sh.at[pl.ds(r, ZR)],
                      out_h.at[pl.ds(cid * N + r, ZR)])
      return carry

    lax.fori_loop(0, n_my_zero, ob, 0)

    @pl.when(sid == 0)
    def _():
      pltpu.sync_copy(accd_sh, outd_h.at[pl.ds(cid * DROWS, DROWS)])

  return ek(q, kv, w21, srcme, dst)


def _meta_block(mnf, mei, wq, wk, wv, wo, wm1, wm2):
  def body(mei_ref, mnf_ref, wq_ref, wk_ref, wv_ref, wo_ref, wm1_ref,
           wm2_ref, np2_ref, np1_ref, ep21_ref):
    f = mnf_ref[...]
    iota_n = lax.broadcasted_iota(jnp.int32, (ME, MN), 1)
    msrc = mei_ref[0, :].reshape(ME, 1)
    mdst = mei_ref[1, :].reshape(ME, 1)
    oh_s = (msrc == iota_n).astype(jnp.float32)
    oh_d = (mdst == iota_n).astype(jnp.float32)
    mm = functools.partial(jnp.dot, preferred_element_type=jnp.float32)
    q = mm(f, wq_ref[...])
    kk = mm(f, wk_ref[...])
    vv = mm(f, wv_ref[...])
    q_d = mm(oh_d, q)
    k_e = mm(oh_s, kk)
    v_e = mm(oh_s, vv)
    logits = jnp.sum(q_d * k_e, axis=1, keepdims=True) * INV_SQRT_D
    masked = jnp.where(oh_d > 0.0, logits, -1e30)
    m = jnp.max(masked, axis=0, keepdims=True)
    m = jnp.where(m < -1e29, 0.0, m)
    ex = jnp.exp(logits - mm(oh_d, m.reshape(MN, 1)))
    ssum = lax.dot_general(oh_d, ex, (((0,), (0,)), ((), ())),
                           preferred_element_type=jnp.float32)
    alpha = ex / (mm(oh_d, ssum) + 1e-9)
    msg = alpha * v_e
    segsum = lax.dot_general(oh_d, msg, (((0,), (0,)), ((), ())),
                             preferred_element_type=jnp.float32)
    mn_out = jnp.maximum(mm(segsum, wo_ref[...]), 0.0)
    np2_ref[...] = jnp.tanh(mm(mn_out, wm2_ref[...]))
    np1_ref[...] = jnp.tanh(mm(mn_out, wm1_ref[...]))
    ep21_ref[:, :D] = jnp.tanh(mm(msg, wm2_ref[...]))
    ep21_ref[:, D:] = jnp.tanh(mm(msg, wm1_ref[...]))

  out = [jax.ShapeDtypeStruct((MN, D), jnp.float32),
         jax.ShapeDtypeStruct((MN, D), jnp.float32),
         jax.ShapeDtypeStruct((ME, 2 * D), jnp.float32)]
  return pl.pallas_call(body, out_shape=out)(
      mei, mnf, wq, wk, wv, wo, wm1, wm2)


_TILE = 400
_NT = N // _TILE


def _node_params(mid3, np2, np1):
  def body(mid_ref, np2_ref, np1_ref, w2_ref, w1_ref):
    mid = mid_ref[0, 0, :].reshape(_TILE, 1)
    iota_n = lax.broadcasted_iota(jnp.int32, (_TILE, MN), 1)
    oh = (mid == iota_n).astype(jnp.float32)
    mm = functools.partial(jnp.dot, preferred_element_type=jnp.float32)
    w2_ref[...] = mm(oh, np2_ref[...])
    w1_ref[...] = mm(oh, np1_ref[...])

  out = [jax.ShapeDtypeStruct((N, D), jnp.float32),
         jax.ShapeDtypeStruct((N, D), jnp.float32)]
  return pl.pallas_call(
      body,
      grid=(_NT,),
      in_specs=[pl.BlockSpec((1, 1, _TILE), lambda i: (i, 0, 0)),
                pl.BlockSpec((MN, D), lambda i: (0, 0)),
                pl.BlockSpec((MN, D), lambda i: (0, 0))],
      out_specs=[pl.BlockSpec((_TILE, D), lambda i: (i, 0)),
                 pl.BlockSpec((_TILE, D), lambda i: (i, 0))],
      out_shape=out)(mid3, np2, np1)


def _qkv(feat, wq, wk, wv, w2n):
  def body(f_ref, wq_ref, wk_ref, wv_ref, w2_ref, q_ref, kv_ref):
    f = f_ref[...]
    mm = functools.partial(jnp.dot, preferred_element_type=jnp.float32)
    q_ref[...] = mm(f, wq_ref[...]) * (1.0 + w2_ref[...])
    kv_ref[:, :D] = mm(f, wk_ref[...])
    kv_ref[:, D:] = mm(f, wv_ref[...])

  out = [jax.ShapeDtypeStruct((N, D), jnp.float32),
         jax.ShapeDtypeStruct((N, 2 * D), jnp.float32)]
  tile = pl.BlockSpec((_TILE, D), lambda i: (i, 0))
  tile2 = pl.BlockSpec((_TILE, 2 * D), lambda i: (i, 0))
  full = pl.BlockSpec((D, D), lambda i: (0, 0))
  return pl.pallas_call(
      body, grid=(_NT,),
      in_specs=[tile, full, full, full, tile],
      out_specs=[tile, tile2],
      out_shape=out)(feat, wq, wk, wv, w2n)


def _combine(acc0, acc1, den0_3, den1_3, wo, w1n):
  def body(a0_ref, a1_ref, d0_ref, d1_ref, wo_ref, w1_ref, o_ref):
    num = a0_ref[...] + a1_ref[...]
    den = (d0_ref[0, 0, :] + d1_ref[0, 0, :]).reshape(_TILE, 1)
    pre = num / (den + 1e-9)
    mm = functools.partial(jnp.dot, preferred_element_type=jnp.float32)
    o_ref[...] = jnp.maximum(mm(pre, wo_ref[...]) + w1_ref[...], 0.0)

  tile = pl.BlockSpec((_TILE, D), lambda i: (i, 0))
  dtile = pl.BlockSpec((1, 1, _TILE), lambda i: (i, 0, 0))
  full = pl.BlockSpec((D, D), lambda i: (0, 0))
  return pl.pallas_call(
      body, grid=(_NT,),
      in_specs=[tile, tile, dtile, dtile, full, tile],
      out_specs=tile,
      out_shape=jax.ShapeDtypeStruct((N, D), jnp.float32))(
          acc0, acc1, den0_3, den1_3, wo, w1n)


def kernel(node_values, edge_index, meta_node_values, meta_edge_index,
           meta_node_id, meta_edge_id, emb_table, Wm1, Wm2, Wq_meta,
           Wk_meta, Wv_meta, Wo_meta, Wq, Wk, Wv, Wo):
  nv = node_values.astype(jnp.int32)
  mnv = meta_node_values.astype(jnp.int32)
  total = N + MN
  padded = ((total + 8 * NW * 10 - 1) // (8 * NW * 10)) * (8 * NW * 10)
  idx_all = jnp.concatenate(
      [nv, mnv, jnp.zeros((padded - total,), jnp.int32)])
  rows = _emb_gather(emb_table.astype(jnp.float32), idx_all)
  node_feat = rows[:N]
  mn_feat = rows[N:N + MN]

  np2, np1, ep21 = _meta_block(
      mn_feat, meta_edge_index.astype(jnp.int32),
      Wq_meta[0], Wk_meta[0], Wv_meta[0], Wo_meta[0], Wm1, Wm2)

  mid3 = meta_node_id.astype(jnp.int32).reshape(_NT, 1, _TILE)
  w2n, w1n = _node_params(mid3, np2, np1)

  src = edge_index[0].astype(jnp.int32)
  dst = edge_index[1].astype(jnp.int32)
  me = meta_edge_id.astype(jnp.int32)
  srcme = jnp.concatenate(
      [src.reshape(-1, 40), me.reshape(-1, 40)], axis=1).reshape(-1)

  feats = []
  feat = node_feat
  for i in range(2):
    q, kv = _qkv(feat, Wq[i], Wk[i], Wv[i], w2n)
    acc, accd = _edge_pass(q, kv, ep21, srcme, dst)
    den0_3 = accd[:DROWS].reshape(DROWS * D)[:N].reshape(_NT, 1, _TILE)
    den1_3 = accd[DROWS:].reshape(DROWS * D)[:N].reshape(_NT, 1, _TILE)
    feat = _combine(acc[:N], acc[N:], den0_3, den1_3, Wo[i], w1n)
    feats.append(feat)
  return jnp.stack(feats)
